# baseline (speedup 1.0000x reference)
import functools

import jax
import jax.numpy as jnp
from jax import lax
from jax.experimental import pallas as pl
from jax.experimental.pallas import tpu as pltpu
from jax.experimental.pallas import tpu_sc as plsc

_D = 128
_CH = 128


def _embed(ids2d, table):
    n_rows = ids2d.shape[0] * ids2d.shape[1]
    info = plsc.get_sparse_core_info()
    nc = info.num_cores
    nw = nc * info.num_subcores
    rows_w = n_rows // nw
    nch = rows_w // _CH
    irows_w = ids2d.shape[0] // nw

    mesh = plsc.VectorSubcoreMesh(core_axis_name="c", subcore_axis_name="s")

    @functools.partial(
        pl.kernel,
        mesh=mesh,
        out_type=jax.ShapeDtypeStruct((n_rows, _D), jnp.float32),
        scratch_types=[
            pltpu.VMEM((irows_w, 128), jnp.int32),
            pltpu.VMEM((_CH, _D), jnp.float32),
            pltpu.VMEM((_CH, _D), jnp.float32),
            pltpu.SemaphoreType.DMA,
            pltpu.SemaphoreType.DMA,
        ],
    )
    def emb(ids_hbm, table_hbm, out_hbm, idx_v, buf0, buf1, gsem, ssem):
        wid = lax.axis_index("s") * nc + lax.axis_index("c")
        base = wid * rows_w
        pltpu.sync_copy(ids_hbm.at[pl.ds(wid * irows_w, irows_w)], idx_v)

        bufs = (buf0, buf1)

        def run_chunk(c, buf):
            pltpu.async_copy(table_hbm.at[idx_v.at[c]], buf, gsem).wait()
            pltpu.async_copy(buf, out_hbm.at[pl.ds(base + c * _CH, _CH)], ssem)

        for b in range(2):
            run_chunk(b, bufs[b])

        def body(g, carry):
            for b in range(2):
                pltpu.make_async_copy(
                    bufs[b], out_hbm.at[pl.ds(base, _CH)], ssem).wait()
                run_chunk(2 * g + b, bufs[b])
            return carry

        lax.fori_loop(1, nch // 2, body, 0)

        for b in range(2):
            pltpu.make_async_copy(
                bufs[b], out_hbm.at[pl.ds(base, _CH)], ssem).wait()

    return emb(ids2d, table)


def kernel(input_ids, table):
    b, l = input_ids.shape
    ids2d = input_ids.astype(jnp.int32).reshape(-1, 128)
    out = _embed(ids2d, table)
    return out.reshape(b, l, _D)


# speedup vs baseline: 1.2173x; 1.2173x over previous
import functools

import jax
import jax.numpy as jnp
from jax import lax
from jax.experimental import pallas as pl
from jax.experimental.pallas import tpu as pltpu
from jax.experimental.pallas import tpu_sc as plsc

_D = 128
_CH = 256
_IR = _CH // 128


def _embed(ids, table):
    n_rows = ids.shape[0]
    info = plsc.get_sparse_core_info()
    nc = info.num_cores
    nw = nc * info.num_subcores
    rows_w = n_rows // nw
    nch = rows_w // _CH

    mesh = plsc.VectorSubcoreMesh(core_axis_name="c", subcore_axis_name="s")

    @functools.partial(
        pl.kernel,
        mesh=mesh,
        out_type=jax.ShapeDtypeStruct((n_rows, _D), jnp.float32),
        scratch_types=[
            pltpu.VMEM((rows_w,), jnp.int32),
            pltpu.VMEM((_CH, _D), jnp.float32),
            pltpu.VMEM((_CH, _D), jnp.float32),
            pltpu.SemaphoreType.DMA,
            pltpu.SemaphoreType.DMA,
        ],
    )
    def emb(ids_hbm, table_hbm, out_hbm, idx_v, buf0, buf1, gsem, ssem):
        wid = lax.axis_index("s") * nc + lax.axis_index("c")
        base = wid * rows_w
        pltpu.sync_copy(ids_hbm.at[pl.ds(base, rows_w)], idx_v)

        bufs = (buf0, buf1)

        def run_chunk(c, buf):
            idx = idx_v.at[pl.ds(c * _CH, _CH)]
            pltpu.async_copy(table_hbm.at[idx], buf, gsem).wait()
            pltpu.async_copy(buf, out_hbm.at[pl.ds(base + c * _CH, _CH)], ssem)

        for b in range(2):
            run_chunk(b, bufs[b])

        def body(g, carry):
            for b in range(2):
                pltpu.make_async_copy(
                    bufs[b], out_hbm.at[pl.ds(base, _CH)], ssem).wait()
                run_chunk(2 * g + b, bufs[b])
            return carry

        lax.fori_loop(1, nch // 2, body, 0)

        for b in range(2):
            pltpu.make_async_copy(
                bufs[b], out_hbm.at[pl.ds(base, _CH)], ssem).wait()

    return emb(ids, table)


def kernel(input_ids, table):
    b, l = input_ids.shape
    ids = input_ids.astype(jnp.int32).reshape(-1)
    out = _embed(ids, table)
    return out.reshape(b, l, _D)


# speedup vs baseline: 1.2223x; 1.0041x over previous
import functools

import jax
import jax.numpy as jnp
from jax import lax
from jax.experimental import pallas as pl
from jax.experimental.pallas import tpu as pltpu
from jax.experimental.pallas import tpu_sc as plsc

_D = 128
_CH = 200
_NB = 4


def _embed(ids, table):
    n_rows = ids.shape[0]
    info = plsc.get_sparse_core_info()
    nc = info.num_cores
    nw = nc * info.num_subcores
    rows_w = n_rows // nw
    nch = rows_w // _CH

    mesh = plsc.VectorSubcoreMesh(core_axis_name="c", subcore_axis_name="s")

    @functools.partial(
        pl.kernel,
        mesh=mesh,
        out_type=jax.ShapeDtypeStruct((n_rows, _D), jnp.float32),
        scratch_types=[
            pltpu.VMEM((rows_w,), jnp.int32),
        ] + [pltpu.VMEM((_CH, _D), jnp.float32) for _ in range(_NB)]
          + [pltpu.SemaphoreType.DMA for _ in range(2 * _NB)],
    )
    def emb(ids_hbm, table_hbm, out_hbm, idx_v, *rest):
        bufs = rest[:_NB]
        gs = rest[_NB:2 * _NB]
        ss = rest[2 * _NB:]

        wid = lax.axis_index("s") * nc + lax.axis_index("c")
        base = wid * rows_w
        pltpu.sync_copy(ids_hbm.at[pl.ds(base, rows_w)], idx_v)

        def gather(c, b):
            idx = idx_v.at[pl.ds(c * _CH, _CH)]
            pltpu.async_copy(table_hbm.at[idx], bufs[b], gs[b])

        def wait_gather(b):
            pltpu.make_async_copy(
                table_hbm.at[pl.ds(0, _CH)], bufs[b], gs[b]).wait()

        def store(c, b):
            pltpu.async_copy(
                bufs[b], out_hbm.at[pl.ds(base + c * _CH, _CH)], ss[b])

        def wait_store(b):
            pltpu.make_async_copy(
                bufs[b], out_hbm.at[pl.ds(base, _CH)], ss[b]).wait()

        def step(c, j, head, tail):
            wait_gather(j)
            store(c, j)
            if tail:
                return
            j2 = (j + 2) % _NB
            if not head:
                wait_store(j2)
            gather(c + 2, j2)

        gather(0, 0)
        gather(1, 1)
        for j in range(_NB):
            step(j, j, head=(j < 2), tail=False)

        def body(g, carry):
            for j in range(_NB):
                step(g * _NB + j, j, head=False, tail=False)
            return carry

        lax.fori_loop(1, nch // _NB - 1, body, 0)

        for j in range(_NB):
            c = nch - _NB + j
            step(c, j, head=False, tail=(j >= 2))

        for j in range(_NB):
            wait_store(j)

    return emb(ids, table)


def kernel(input_ids, table):
    b, l = input_ids.shape
    ids = input_ids.astype(jnp.int32).reshape(-1)
    out = _embed(ids, table)
    return out.reshape(b, l, _D)
